# baseline (speedup 1.0000x reference)
import functools

import jax
import jax.numpy as jnp
from jax import lax
from jax.experimental import pallas as pl
from jax.experimental.pallas import tpu as pltpu
from jax.experimental.pallas import tpu_sc as plsc

B, L, H = 1024, 200, 128
N = B * L
NW = 32
S = 4
BS = B // S
NS = BS * L
PER_W = NS // NW
CHUNK = 200
N_CHUNKS = PER_W // CHUNK
BB = 64
GS = BS // BB


def _gather_sc(table, ids_flat):
  mesh = plsc.VectorSubcoreMesh(core_axis_name="c", subcore_axis_name="s")

  @functools.partial(
      pl.kernel,
      mesh=mesh,
      out_type=jax.ShapeDtypeStruct((NS, H), jnp.float32),
      scratch_types=[
          pltpu.VMEM((PER_W,), jnp.int32),
          pltpu.VMEM((CHUNK, H), jnp.float32),
          pltpu.VMEM((CHUNK, H), jnp.float32),
          pltpu.SemaphoreType.DMA,
          pltpu.SemaphoreType.DMA,
          pltpu.SemaphoreType.DMA,
          pltpu.SemaphoreType.DMA,
      ],
  )
  def k(table_hbm, ids_hbm, out_hbm, idx_v, rows0, rows1, sg0, sg1, sw0, sw1):
    wid = lax.axis_index("s") * 2 + lax.axis_index("c")
    wbase = wid * PER_W
    pltpu.sync_copy(ids_hbm.at[pl.ds(wbase, PER_W)], idx_v)

    rows = (rows0, rows1)
    sg = (sg0, sg1)
    sw = (sw0, sw1)
    hg = [None, None]
    hw = [None, None]
    for i in range(N_CHUNKS):
      p = i % 2
      if i >= 2:
        hw[p].wait()
      hg[p] = pltpu.async_copy(
          table_hbm.at[idx_v.at[pl.ds(i * CHUNK, CHUNK)]], rows[p], sg[p])
      if i >= 1:
        q = 1 - p
        hg[q].wait()
        hw[q] = pltpu.async_copy(
            rows[q], out_hbm.at[pl.ds(wbase + (i - 1) * CHUNK, CHUNK)], sw[q])
    last = N_CHUNKS - 1
    p = last % 2
    hg[p].wait()
    hw[p] = pltpu.async_copy(
        rows[p], out_hbm.at[pl.ds(wbase + last * CHUNK, CHUNK)], sw[p])
    hw[1 - p].wait()
    hw[p].wait()

  return k(table, ids_flat)


def _ln_math(x_ref, tt_ref, posty_ref, g_ref, b_ref, o_ref):
  x = x_ref[...]
  ttf = tt_ref[:, 0, :].astype(jnp.float32)
  pt0 = posty_ref[0]
  dty = posty_ref[1]
  x = x + pt0[None] + ttf[:, :, None] * dty[None]
  mean = jnp.mean(x, axis=-1, keepdims=True)
  var = jnp.mean(jnp.square(x - mean), axis=-1, keepdims=True)
  y = (x - mean) * lax.rsqrt(var + 1e-5)
  o_ref[...] = y * g_ref[0, :][None, None, :] + b_ref[0, :][None, None, :]


def _ln_body_first(x_ref, tt_ref, posty_ref, g_ref, b_ref, o_ref):
  _ln_math(x_ref, tt_ref, posty_ref, g_ref, b_ref, o_ref)


def _ln_body_chained(prev_ref, x_ref, tt_ref, posty_ref, g_ref, b_ref, o_ref):
  del prev_ref
  _ln_math(x_ref, tt_ref, posty_ref, g_ref, b_ref, o_ref)


def _ln_slice(prev, x, tt3, posty, gamma2, beta2, s):
  soff = s * GS
  data_specs = [
      pl.BlockSpec((BB, L, H), lambda i: (i, 0, 0)),
      pl.BlockSpec((BB, 1, L), lambda i: (i, 0, 0)),
      pl.BlockSpec((2, L, H), lambda i: (0, 0, 0)),
      pl.BlockSpec((1, H), lambda i: (0, 0)),
      pl.BlockSpec((1, H), lambda i: (0, 0)),
  ]
  out_spec = pl.BlockSpec((BB, L, H), lambda i: (soff + i, 0, 0))
  out_shape = jax.ShapeDtypeStruct((B, L, H), jnp.float32)
  if prev is None:
    return pl.pallas_call(
        _ln_body_first,
        grid=(GS,),
        in_specs=data_specs,
        out_specs=out_spec,
        out_shape=out_shape,
    )(x, tt3, posty, gamma2, beta2)
  return pl.pallas_call(
      _ln_body_chained,
      grid=(GS,),
      in_specs=[pl.BlockSpec(memory_space=pltpu.MemorySpace.HBM)] + data_specs,
      out_specs=out_spec,
      out_shape=out_shape,
      input_output_aliases={0: 0},
  )(prev, x, tt3, posty, gamma2, beta2)


def kernel(input_ids, token_type_ids, token_table, pos_table, type_table,
           ln_gamma, ln_beta):
  ids = input_ids.reshape(S, NS).astype(jnp.int32)
  tt4 = token_type_ids.reshape(S, BS, 1, L).astype(jnp.int32)
  pos_eff = pos_table[:L] + type_table[0][None, :]
  dty = jnp.broadcast_to(type_table[1] - type_table[0], (L, H))
  posty = jnp.stack([pos_eff, dty])
  gamma2 = ln_gamma.reshape(1, H)
  beta2 = ln_beta.reshape(1, H)

  temps = [_gather_sc(token_table, ids[s]) for s in range(S)]
  out = None
  for s in range(S):
    x = temps[s].reshape(BS, L, H)
    out = _ln_slice(out, x, tt4[s], posty, gamma2, beta2, s)
  return out


# speedup vs baseline: 1.0399x; 1.0399x over previous
import functools

import jax
import jax.numpy as jnp
from jax import lax
from jax.experimental import pallas as pl
from jax.experimental.pallas import tpu as pltpu
from jax.experimental.pallas import tpu_sc as plsc

B, L, H = 1024, 200, 128
N = B * L
NW = 32
S = 4
BS = B // S
NS = BS * L
PER_W = NS // NW
CHUNK = 400
N_CHUNKS = PER_W // CHUNK
BB = 64
GS = BS // BB


def _gather_sc(table, ids_flat):
  mesh = plsc.VectorSubcoreMesh(core_axis_name="c", subcore_axis_name="s")

  @functools.partial(
      pl.kernel,
      mesh=mesh,
      out_type=jax.ShapeDtypeStruct((NS, H), jnp.float32),
      scratch_types=[
          pltpu.VMEM((PER_W,), jnp.int32),
          pltpu.VMEM((CHUNK, H), jnp.float32),
          pltpu.VMEM((CHUNK, H), jnp.float32),
          pltpu.SemaphoreType.DMA,
          pltpu.SemaphoreType.DMA,
          pltpu.SemaphoreType.DMA,
          pltpu.SemaphoreType.DMA,
      ],
  )
  def k(table_hbm, ids_hbm, out_hbm, idx_v, rows0, rows1, sg0, sg1, sw0, sw1):
    wid = lax.axis_index("s") * 2 + lax.axis_index("c")
    wbase = wid * PER_W
    pltpu.sync_copy(ids_hbm.at[pl.ds(wbase, PER_W)], idx_v)

    rows = (rows0, rows1)
    sg = (sg0, sg1)
    sw = (sw0, sw1)
    hg = [None, None]
    hw = [None, None]
    for i in range(N_CHUNKS):
      p = i % 2
      if i >= 2:
        hw[p].wait()
      hg[p] = pltpu.async_copy(
          table_hbm.at[idx_v.at[pl.ds(i * CHUNK, CHUNK)]], rows[p], sg[p])
      if i >= 1:
        q = 1 - p
        hg[q].wait()
        hw[q] = pltpu.async_copy(
            rows[q], out_hbm.at[pl.ds(wbase + (i - 1) * CHUNK, CHUNK)], sw[q])
    last = N_CHUNKS - 1
    p = last % 2
    hg[p].wait()
    hw[p] = pltpu.async_copy(
        rows[p], out_hbm.at[pl.ds(wbase + last * CHUNK, CHUNK)], sw[p])
    hw[1 - p].wait()
    hw[p].wait()

  return k(table, ids_flat)


def _ln_math(x_ref, tt_ref, posty_ref, g_ref, b_ref, o_ref):
  x = x_ref[...]
  ttf = tt_ref[:, 0, :].astype(jnp.float32)
  pt0 = posty_ref[0]
  dty = posty_ref[1]
  x = x + pt0[None] + ttf[:, :, None] * dty[None]
  mean = jnp.mean(x, axis=-1, keepdims=True)
  var = jnp.mean(jnp.square(x - mean), axis=-1, keepdims=True)
  y = (x - mean) * lax.rsqrt(var + 1e-5)
  o_ref[...] = y * g_ref[0, :][None, None, :] + b_ref[0, :][None, None, :]


def _ln_body_first(x_ref, tt_ref, posty_ref, g_ref, b_ref, o_ref):
  _ln_math(x_ref, tt_ref, posty_ref, g_ref, b_ref, o_ref)


def _ln_body_chained(prev_ref, x_ref, tt_ref, posty_ref, g_ref, b_ref, o_ref):
  del prev_ref
  _ln_math(x_ref, tt_ref, posty_ref, g_ref, b_ref, o_ref)


def _ln_slice(prev, x, tt3, posty, gamma2, beta2, s):
  soff = s * GS
  data_specs = [
      pl.BlockSpec((BB, L, H), lambda i: (i, 0, 0)),
      pl.BlockSpec((BB, 1, L), lambda i: (i, 0, 0)),
      pl.BlockSpec((2, L, H), lambda i: (0, 0, 0)),
      pl.BlockSpec((1, H), lambda i: (0, 0)),
      pl.BlockSpec((1, H), lambda i: (0, 0)),
  ]
  out_spec = pl.BlockSpec((BB, L, H), lambda i: (soff + i, 0, 0))
  out_shape = jax.ShapeDtypeStruct((B, L, H), jnp.float32)
  if prev is None:
    return pl.pallas_call(
        _ln_body_first,
        grid=(GS,),
        in_specs=data_specs,
        out_specs=out_spec,
        out_shape=out_shape,
    )(x, tt3, posty, gamma2, beta2)
  return pl.pallas_call(
      _ln_body_chained,
      grid=(GS,),
      in_specs=[pl.BlockSpec(memory_space=pltpu.MemorySpace.HBM)] + data_specs,
      out_specs=out_spec,
      out_shape=out_shape,
      input_output_aliases={0: 0},
  )(prev, x, tt3, posty, gamma2, beta2)


def kernel(input_ids, token_type_ids, token_table, pos_table, type_table,
           ln_gamma, ln_beta):
  ids = input_ids.reshape(S, NS).astype(jnp.int32)
  tt4 = token_type_ids.reshape(S, BS, 1, L).astype(jnp.int32)
  pos_eff = pos_table[:L] + type_table[0][None, :]
  dty = jnp.broadcast_to(type_table[1] - type_table[0], (L, H))
  posty = jnp.stack([pos_eff, dty])
  gamma2 = ln_gamma.reshape(1, H)
  beta2 = ln_beta.reshape(1, H)

  temps = [_gather_sc(token_table, ids[s]) for s in range(S)]
  out = None
  for s in range(S):
    x = temps[s].reshape(BS, L, H)
    out = _ln_slice(out, x, tt4[s], posty, gamma2, beta2, s)
  return out
